# baseline (speedup 1.0000x reference)
import functools

import jax
import jax.numpy as jnp
from jax import lax
from jax.experimental import pallas as pl
from jax.experimental.pallas import tpu as pltpu
from jax.experimental.pallas import tpu_sc as plsc

N_FEATURES = 40960
D = 257
DH = 256
BATCH = 16384
K = 32
NBAGS = 2 * BATCH
NC, NS = 2, 16
NW = NC * NS
BAGS_PER_W = NBAGS // NW
BCH = 128
NCH = BAGS_PER_W // BCH
GROWS = 128
GBAGS = GROWS // K
NG = BCH // GBAGS
SL = 2
L = 16


def _mesh():
    return plsc.VectorSubcoreMesh(
        core_axis_name="c", subcore_axis_name="s", num_cores=NC, num_subcores=NS
    )


@functools.cache
def _make_embed_bag():
    return functools.partial(
        pl.kernel,
        out_type=(
            jax.ShapeDtypeStruct((NBAGS, DH // 2), jnp.int32),
            jax.ShapeDtypeStruct((NBAGS,), jnp.float32),
        ),
        mesh=_mesh(),
        compiler_params=pltpu.CompilerParams(needs_layout_passes=False),
        scratch_types=[
            pltpu.VMEM((BCH * K,), jnp.int32),
            pltpu.VMEM((K, BCH), jnp.int32),
            pltpu.VMEM((2, GROWS, DH // 2), jnp.int32),
            pltpu.VMEM((BCH, DH // 2), jnp.int32),
            pltpu.VMEM((N_FEATURES,), jnp.float32),
            pltpu.VMEM((BCH,), jnp.float32),
            pltpu.SemaphoreType.DMA,
            pltpu.SemaphoreType.DMA,
        ],
    )(_embed_bag_body)


def _embed_bag_body(icsf_hbm, icst_hbm, table_hbm, psqt_hbm,
                    out_hbm, outp_hbm,
                    idxf, idxt, rows, acc, psqt_v, pacc, sem0, sem1):
    wid = lax.axis_index("s") * NC + lax.axis_index("c")
    pltpu.sync_copy(psqt_hbm, psqt_v)
    sems = (sem0, sem1)

    def fire(g, buf):
        pltpu.async_copy(
            table_hbm.at[idxf.at[pl.ds(g * GROWS, GROWS)]],
            rows.at[buf], sems[buf])

    def wait(buf):
        pltpu.make_async_copy(
            table_hbm.at[idxf.at[pl.ds(0, GROWS)]],
            rows.at[buf], sems[buf]).wait()

    def reduce_buf(g, buf):
        rb = rows.at[buf]

        def ld(r, s):
            return plsc.bitcast(rb[r, s], jnp.bfloat16)

        def bag_body(bag, c1):
            r0 = bag * K
            for cb in range(DH // 2 // L):
                s = pl.ds(cb * L, L)
                a = [ld(r0 + i, s) + ld(r0 + 8 + i, s) for i in range(8)]
                for i in range(8):
                    a[i] = a[i] + ld(r0 + 16 + i, s)
                    a[i] = a[i] + ld(r0 + 24 + i, s)
                b0 = (a[0] + a[1]) + (a[2] + a[3])
                b1 = (a[4] + a[5]) + (a[6] + a[7])
                acc[g * GBAGS + bag, s] = plsc.bitcast(b0 + b1, jnp.int32)
            return c1

        lax.fori_loop(0, GBAGS, bag_body, 0)

    def chunk_body(ci, carry):
        base = wid * BAGS_PER_W + ci * BCH
        pltpu.sync_copy(icsf_hbm.at[pl.ds(base * K, BCH * K)], idxf)
        pltpu.sync_copy(icst_hbm.at[:, pl.ds(base, BCH)], idxt)
        fire(0, 0)
        fire(1, 1)

        for i in range(BCH // L):
            pacc[pl.ds(i * L, L)] = jnp.zeros((L,), jnp.float32)

        def psum(j, c1):
            for i in range(BCH // L):
                s = pl.ds(i * L, L)
                pacc[s] += plsc.load_gather(psqt_v, [idxt[j, s]])
            return c1

        lax.fori_loop(0, K, psum, 0)

        def pipe_body(g2, c1):
            g = 2 * g2
            wait(0)
            reduce_buf(g, 0)

            @pl.when(g2 != NG // 2 - 1)
            def _():
                fire(g + 2, 0)

            wait(1)
            reduce_buf(g + 1, 1)

            @pl.when(g2 != NG // 2 - 1)
            def _():
                fire(g + 3, 1)

            return c1

        lax.fori_loop(0, NG // 2, pipe_body, 0)
        pltpu.sync_copy(acc, out_hbm.at[pl.ds(base, BCH)])
        pltpu.sync_copy(pacc, outp_hbm.at[pl.ds(base, BCH)])
        return carry

    lax.fori_loop(0, NCH, chunk_body, 0)


def _fc_body(w_ref, b_ref, wp_ref, bp_ref, stm_ref, bias_ref, fcw_ref,
             fcb_ref, out_ref):
    bias = bias_ref[...]
    wfts = w_ref[...].astype(jnp.float32) + bias
    bfts = b_ref[...].astype(jnp.float32) + bias
    s = stm_ref[...]
    x1 = (1.0 - s) * wfts + s * bfts
    x2 = (1.0 - s) * bfts + s * wfts
    fcw = fcw_ref[...]
    fca, fcbb = fcw[:, :DH], fcw[:, DH:]
    acc = jnp.sum(jnp.clip(x1, 0.0, 1.0) * fca, axis=1, keepdims=True)
    acc = acc + jnp.sum(jnp.clip(x2, 0.0, 1.0) * fcbb, axis=1, keepdims=True)
    out_ref[...] = acc + fcb_ref[...] + (wp_ref[...] - bp_ref[...]) * (0.5 - s)


def kernel(wft_ics, bft_ics, stm, ft_weight, ft_bias, fc_w, fc_b):
    ics = jnp.concatenate([wft_ics, bft_ics], axis=0)
    ics_flat = ics.reshape(-1)
    ics_t = ics.T
    psqt_col = ft_weight[:, DH]
    tb = ft_weight[:, :DH].astype(jnp.bfloat16)
    t32 = lax.bitcast_convert_type(
        tb.reshape(N_FEATURES, DH // 2, 2), jnp.int32)
    acc32, psqt = _make_embed_bag()(ics_flat, ics_t, t32, psqt_col)
    acc = lax.bitcast_convert_type(acc32, jnp.bfloat16).reshape(NBAGS, DH)
    psqt2 = psqt.reshape(NBAGS, 1)

    bm = 512
    nb = BATCH // bm
    out = pl.pallas_call(
        _fc_body,
        grid=(nb,),
        in_specs=[
            pl.BlockSpec((bm, DH), lambda i: (i, 0)),
            pl.BlockSpec((bm, DH), lambda i: (i + nb, 0)),
            pl.BlockSpec((bm, 1), lambda i: (i, 0)),
            pl.BlockSpec((bm, 1), lambda i: (i + nb, 0)),
            pl.BlockSpec((bm, 1), lambda i: (i, 0)),
            pl.BlockSpec((1, DH), lambda i: (0, 0)),
            pl.BlockSpec((1, 512), lambda i: (0, 0)),
            pl.BlockSpec((1, 1), lambda i: (0, 0)),
        ],
        out_specs=pl.BlockSpec((bm, 1), lambda i: (i, 0)),
        out_shape=jax.ShapeDtypeStruct((BATCH, 1), jnp.float32),
    )(acc, acc, psqt2, psqt2, stm, ft_bias[:DH].reshape(1, DH), fc_w,
      fc_b.reshape(1, 1))
    return out


# speedup vs baseline: 1.5112x; 1.5112x over previous
import functools

import jax
import jax.numpy as jnp
from jax import lax
from jax.experimental import pallas as pl
from jax.experimental.pallas import tpu as pltpu
from jax.experimental.pallas import tpu_sc as plsc

N_FEATURES = 40960
D = 257
DH = 256
DP = DH // 2
BATCH = 16384
K = 32
NBAGS = 2 * BATCH
NC, NS = 2, 16
NW = NC * NS
BAGS_PER_W = NBAGS // NW
BCH = 128
NCH = BAGS_PER_W // BCH
GROWS = 128
GBAGS = GROWS // K
NG = BCH // GBAGS
L = 16


def _mesh():
    return plsc.VectorSubcoreMesh(
        core_axis_name="c", subcore_axis_name="s", num_cores=NC, num_subcores=NS
    )


def _pack_body(w_ref, out_ref, psqt_ref):
    x = w_ref[...]
    u = lax.bitcast_convert_type(x[:, :DH], jnp.uint32)
    r = (u + jnp.uint32(0x7FFF) + ((u >> 16) & jnp.uint32(1))) >> 16
    lo, hi = r[:, :DP], r[:, DP:]
    out_ref[...] = lax.bitcast_convert_type(lo | (hi << 16), jnp.int32)
    psqt_ref[...] = x[:, DH:]


@functools.cache
def _make_pack():
    blk = 4096
    return pl.pallas_call(
        _pack_body,
        grid=(N_FEATURES // blk,),
        in_specs=[pl.BlockSpec((blk, D), lambda i: (i, 0))],
        out_specs=[
            pl.BlockSpec((blk, DP), lambda i: (i, 0)),
            pl.BlockSpec((blk, 1), lambda i: (i, 0)),
        ],
        out_shape=[
            jax.ShapeDtypeStruct((N_FEATURES, DP), jnp.int32),
            jax.ShapeDtypeStruct((N_FEATURES, 1), jnp.float32),
        ],
    )


@functools.cache
def _make_embed_bag():
    return functools.partial(
        pl.kernel,
        out_type=(
            jax.ShapeDtypeStruct((NBAGS, DP), jnp.int32),
            jax.ShapeDtypeStruct((NBAGS,), jnp.float32),
        ),
        mesh=_mesh(),
        compiler_params=pltpu.CompilerParams(needs_layout_passes=False),
        scratch_types=[
            pltpu.VMEM((BCH * K,), jnp.int32),
            pltpu.VMEM((2, GROWS, DP), jnp.int32),
            pltpu.VMEM((BCH, DP), jnp.int32),
            pltpu.VMEM((N_FEATURES,), jnp.float32),
            pltpu.VMEM((BCH,), jnp.float32),
            pltpu.SemaphoreType.DMA,
            pltpu.SemaphoreType.DMA,
        ],
    )(_embed_bag_body)


def _embed_bag_body(wics_hbm, bics_hbm, table_hbm, psqt_hbm,
                    out_hbm, outp_hbm,
                    idxf, rows, acc, psqt_v, pacc, sem0, sem1):
    wid = lax.axis_index("s") * NC + lax.axis_index("c")
    pltpu.sync_copy(psqt_hbm, psqt_v)
    sems = (sem0, sem1)

    def fire(g, buf):
        pltpu.async_copy(
            table_hbm.at[idxf.at[pl.ds(g * GROWS, GROWS)]],
            rows.at[buf], sems[buf])

    def wait(buf):
        pltpu.make_async_copy(
            table_hbm.at[idxf.at[pl.ds(0, GROWS)]],
            rows.at[buf], sems[buf]).wait()

    def reduce_buf(g, buf):
        rb = rows.at[buf]

        def ld(r, s):
            return plsc.bitcast(rb[r, s], jnp.bfloat16)

        def bag_body(bag, c1):
            r0 = bag * K
            for cb in range(DP // L):
                s = pl.ds(cb * L, L)
                a = [ld(r0 + i, s) + ld(r0 + 8 + i, s) for i in range(8)]
                for i in range(8):
                    a[i] = a[i] + ld(r0 + 16 + i, s)
                    a[i] = a[i] + ld(r0 + 24 + i, s)
                b0 = (a[0] + a[1]) + (a[2] + a[3])
                b1 = (a[4] + a[5]) + (a[6] + a[7])
                acc[g * GBAGS + bag, s] = plsc.bitcast(b0 + b1, jnp.int32)
            return c1

        lax.fori_loop(0, GBAGS, bag_body, 0)

    def chunk_body(ci, carry):
        base = wid * BAGS_PER_W + ci * BCH
        lbase = (wid % NS) * BAGS_PER_W * K + ci * BCH * K

        @pl.when(wid < NS)
        def _():
            pltpu.sync_copy(wics_hbm.at[pl.ds(lbase, BCH * K)], idxf)

        @pl.when(wid >= NS)
        def _():
            pltpu.sync_copy(bics_hbm.at[pl.ds(lbase, BCH * K)], idxf)

        fire(0, 0)
        fire(1, 1)

        lanes = lax.iota(jnp.int32, 16)

        def psum(blk, c1):
            vacc = jnp.zeros((L,), jnp.float32)
            for lane in range(L):
                p0 = (blk * L + lane) * K
                v = plsc.load_gather(psqt_v, [idxf[pl.ds(p0, L)]])
                v = v + plsc.load_gather(psqt_v, [idxf[pl.ds(p0 + L, L)]])
                s = lax.reduce_sum(v, axes=(0,))
                vacc = jnp.where(lanes == lane, s, vacc)
            pacc[pl.ds(blk * L, L)] = vacc
            return c1

        lax.fori_loop(0, BCH // L, psum, 0)

        def pipe_body(g2, c1):
            g = 2 * g2
            wait(0)
            reduce_buf(g, 0)

            @pl.when(g2 != NG // 2 - 1)
            def _():
                fire(g + 2, 0)

            wait(1)
            reduce_buf(g + 1, 1)

            @pl.when(g2 != NG // 2 - 1)
            def _():
                fire(g + 3, 1)

            return c1

        lax.fori_loop(0, NG // 2, pipe_body, 0)
        pltpu.sync_copy(acc, out_hbm.at[pl.ds(base, BCH)])
        pltpu.sync_copy(pacc, outp_hbm.at[pl.ds(base, BCH)])
        return carry

    lax.fori_loop(0, NCH, chunk_body, 0)


def _fc_body(w_ref, b_ref, wp_ref, bp_ref, stm_ref, bias_ref, fcw_ref,
             fcb_ref, out_ref):
    def unpack(v32):
        lo = lax.bitcast_convert_type(v32 << 16, jnp.float32)
        hi = lax.bitcast_convert_type(
            v32 & jnp.int32(-65536), jnp.float32)
        return jnp.concatenate([lo, hi], axis=1)

    bias = bias_ref[...]
    wfts = unpack(w_ref[...]) + bias
    bfts = unpack(b_ref[...]) + bias
    s = stm_ref[...]
    x1 = (1.0 - s) * wfts + s * bfts
    x2 = (1.0 - s) * bfts + s * wfts
    fcw = fcw_ref[...]
    fca, fcbb = fcw[:, :DH], fcw[:, DH:]
    acc = jnp.sum(jnp.clip(x1, 0.0, 1.0) * fca, axis=1, keepdims=True)
    acc = acc + jnp.sum(jnp.clip(x2, 0.0, 1.0) * fcbb, axis=1, keepdims=True)
    out_ref[...] = acc + fcb_ref[...] + (wp_ref[...] - bp_ref[...]) * (0.5 - s)


def kernel(wft_ics, bft_ics, stm, ft_weight, ft_bias, fc_w, fc_b):
    t32, psqt_col = _make_pack()(ft_weight)
    acc32, psqt = _make_embed_bag()(
        wft_ics.reshape(-1), bft_ics.reshape(-1), t32, psqt_col.reshape(-1))
    psqt2 = psqt.reshape(NBAGS, 1)

    bm = 512
    nb = BATCH // bm
    out = pl.pallas_call(
        _fc_body,
        grid=(nb,),
        in_specs=[
            pl.BlockSpec((bm, DP), lambda i: (i, 0)),
            pl.BlockSpec((bm, DP), lambda i: (i + nb, 0)),
            pl.BlockSpec((bm, 1), lambda i: (i, 0)),
            pl.BlockSpec((bm, 1), lambda i: (i + nb, 0)),
            pl.BlockSpec((bm, 1), lambda i: (i, 0)),
            pl.BlockSpec((1, DH), lambda i: (0, 0)),
            pl.BlockSpec((1, 512), lambda i: (0, 0)),
            pl.BlockSpec((1, 1), lambda i: (0, 0)),
        ],
        out_specs=pl.BlockSpec((bm, 1), lambda i: (i, 0)),
        out_shape=jax.ShapeDtypeStruct((BATCH, 1), jnp.float32),
    )(acc32, acc32, psqt2, psqt2, stm, ft_bias[:DH].reshape(1, DH), fc_w,
      fc_b.reshape(1, 1))
    return out


# speedup vs baseline: 1.5464x; 1.0233x over previous
import functools

import jax
import jax.numpy as jnp
from jax import lax
from jax.experimental import pallas as pl
from jax.experimental.pallas import tpu as pltpu
from jax.experimental.pallas import tpu_sc as plsc

N_FEATURES = 40960
D = 257
DH = 256
DP = DH // 2
BATCH = 16384
K = 32
NBAGS = 2 * BATCH
NC, NS = 2, 16
NW = NC * NS
BAGS_PER_W = NBAGS // NW
BCH = 128
NCH = BAGS_PER_W // BCH
GROWS = 128
GBAGS = GROWS // K
NG = BCH // GBAGS
L = 16


def _mesh():
    return plsc.VectorSubcoreMesh(
        core_axis_name="c", subcore_axis_name="s", num_cores=NC, num_subcores=NS
    )


def _pack_body(w_ref, out_ref, psqt_ref):
    x = w_ref[...]
    u = lax.bitcast_convert_type(x[:, :DH], jnp.uint32)
    r = (u + jnp.uint32(0x7FFF) + ((u >> 16) & jnp.uint32(1))) >> 16
    lo, hi = r[:, :DP], r[:, DP:]
    out_ref[...] = lax.bitcast_convert_type(lo | (hi << 16), jnp.int32)
    psqt_ref[...] = x[:, DH:]


@functools.cache
def _make_pack():
    blk = 4096
    return pl.pallas_call(
        _pack_body,
        grid=(N_FEATURES // blk,),
        in_specs=[pl.BlockSpec((blk, D), lambda i: (i, 0))],
        out_specs=[
            pl.BlockSpec((blk, DP), lambda i: (i, 0)),
            pl.BlockSpec((blk, 1), lambda i: (i, 0)),
        ],
        out_shape=[
            jax.ShapeDtypeStruct((N_FEATURES, DP), jnp.int32),
            jax.ShapeDtypeStruct((N_FEATURES, 1), jnp.float32),
        ],
    )


@functools.cache
def _make_embed_bag():
    return functools.partial(
        pl.kernel,
        out_type=(
            jax.ShapeDtypeStruct((NBAGS, DP), jnp.int32),
            jax.ShapeDtypeStruct((NBAGS,), jnp.float32),
        ),
        mesh=_mesh(),
        compiler_params=pltpu.CompilerParams(needs_layout_passes=False),
        scratch_types=[
            pltpu.VMEM((BCH * K,), jnp.int32),
            pltpu.VMEM((2, GROWS, DP), jnp.int32),
            pltpu.VMEM((BCH, DP), jnp.int32),
            pltpu.VMEM((N_FEATURES,), jnp.float32),
            pltpu.VMEM((BCH,), jnp.float32),
            pltpu.SemaphoreType.DMA,
            pltpu.SemaphoreType.DMA,
        ],
    )(_embed_bag_body)


def _embed_bag_body(wics_hbm, bics_hbm, table_hbm, psqt_hbm,
                    out_hbm, outp_hbm,
                    idxf, rows, acc, psqt_v, pacc, sem0, sem1):
    wid = lax.axis_index("s") * NC + lax.axis_index("c")
    pltpu.sync_copy(psqt_hbm, psqt_v)
    sems = (sem0, sem1)

    def fire(g, buf):
        pltpu.async_copy(
            table_hbm.at[idxf.at[pl.ds(g * GROWS, GROWS)]],
            rows.at[buf], sems[buf])

    def wait(buf):
        pltpu.make_async_copy(
            table_hbm.at[idxf.at[pl.ds(0, GROWS)]],
            rows.at[buf], sems[buf]).wait()

    def reduce_buf(g, buf):
        rb = rows.at[buf]

        def ld(r, s):
            return plsc.bitcast(rb[r, s], jnp.bfloat16)

        def bag_body(bag, c1):
            r0 = bag * K
            for cb in range(DP // L):
                s = pl.ds(cb * L, L)
                a = [ld(r0 + i, s) + ld(r0 + 8 + i, s) for i in range(8)]
                for i in range(8):
                    a[i] = a[i] + ld(r0 + 16 + i, s)
                    a[i] = a[i] + ld(r0 + 24 + i, s)
                b0 = (a[0] + a[1]) + (a[2] + a[3])
                b1 = (a[4] + a[5]) + (a[6] + a[7])
                acc[g * GBAGS + bag, s] = plsc.bitcast(b0 + b1, jnp.int32)
            return c1

        lax.fori_loop(0, GBAGS, bag_body, 0)

    def chunk_body(ci, carry):
        base = wid * BAGS_PER_W + ci * BCH
        lbase = (wid % NS) * BAGS_PER_W * K + ci * BCH * K

        @pl.when(wid < NS)
        def _():
            pltpu.sync_copy(wics_hbm.at[pl.ds(lbase, BCH * K)], idxf)

        @pl.when(wid >= NS)
        def _():
            pltpu.sync_copy(bics_hbm.at[pl.ds(lbase, BCH * K)], idxf)

        fire(0, 0)
        fire(1, 1)

        lanes = lax.iota(jnp.int32, 16)

        def psum(blk, c1):
            vacc = jnp.zeros((L,), jnp.float32)
            for lane in range(L):
                p0 = (blk * L + lane) * K
                v = plsc.load_gather(psqt_v, [idxf[pl.ds(p0, L)]])
                v = v + plsc.load_gather(psqt_v, [idxf[pl.ds(p0 + L, L)]])
                s = lax.reduce_sum(v, axes=(0,))
                vacc = jnp.where(lanes == lane, s, vacc)
            pacc[pl.ds(blk * L, L)] = vacc
            return c1

        lax.fori_loop(0, BCH // L, psum, 0)

        def pipe_body(g2, c1):
            g = 2 * g2
            wait(0)
            reduce_buf(g, 0)

            @pl.when(g2 != NG // 2 - 1)
            def _():
                fire(g + 2, 0)

            wait(1)
            reduce_buf(g + 1, 1)

            @pl.when(g2 != NG // 2 - 1)
            def _():
                fire(g + 3, 1)

            return c1

        lax.fori_loop(0, NG // 2, pipe_body, 0)
        pltpu.sync_copy(acc, out_hbm.at[pl.ds(base, BCH)])
        pltpu.sync_copy(pacc, outp_hbm.at[pl.ds(base, BCH)])
        return carry

    lax.fori_loop(0, NCH, chunk_body, 0)


def _fc_body(w_ref, b_ref, wp_ref, bp_ref, stm_ref, bias_ref, fcw_ref,
             fcb_ref, out_ref):
    def unpack(v32):
        lo = lax.bitcast_convert_type(v32 << 16, jnp.float32)
        hi = lax.bitcast_convert_type(
            v32 & jnp.int32(-65536), jnp.float32)
        return jnp.concatenate([lo, hi], axis=1)

    bias = bias_ref[...]
    wfts = unpack(w_ref[...]) + bias
    bfts = unpack(b_ref[...]) + bias
    s = stm_ref[...]
    x1 = (1.0 - s) * wfts + s * bfts
    x2 = (1.0 - s) * bfts + s * wfts
    fcw = fcw_ref[...]
    fca, fcbb = fcw[:, :DH], fcw[:, DH:]
    acc = jnp.sum(jnp.clip(x1, 0.0, 1.0) * fca, axis=1, keepdims=True)
    acc = acc + jnp.sum(jnp.clip(x2, 0.0, 1.0) * fcbb, axis=1, keepdims=True)
    out_ref[...] = acc + fcb_ref[...] + (wp_ref[...] - bp_ref[...]) * (0.5 - s)


def kernel(wft_ics, bft_ics, stm, ft_weight, ft_bias, fc_w, fc_b):
    t32, psqt_col = _make_pack()(ft_weight)
    acc32, psqt = _make_embed_bag()(
        wft_ics.reshape(-1), bft_ics.reshape(-1), t32, psqt_col.reshape(-1))
    psqt2 = psqt.reshape(NBAGS, 1)

    bm = 2048
    nb = BATCH // bm
    out = pl.pallas_call(
        _fc_body,
        grid=(nb,),
        in_specs=[
            pl.BlockSpec((bm, DP), lambda i: (i, 0)),
            pl.BlockSpec((bm, DP), lambda i: (i + nb, 0)),
            pl.BlockSpec((bm, 1), lambda i: (i, 0)),
            pl.BlockSpec((bm, 1), lambda i: (i + nb, 0)),
            pl.BlockSpec((bm, 1), lambda i: (i, 0)),
            pl.BlockSpec((1, DH), lambda i: (0, 0)),
            pl.BlockSpec((1, 512), lambda i: (0, 0)),
            pl.BlockSpec((1, 1), lambda i: (0, 0)),
        ],
        out_specs=pl.BlockSpec((bm, 1), lambda i: (i, 0)),
        out_shape=jax.ShapeDtypeStruct((BATCH, 1), jnp.float32),
    )(acc32, acc32, psqt2, psqt2, stm, ft_bias[:DH].reshape(1, DH), fc_w,
      fc_b.reshape(1, 1))
    return out


# speedup vs baseline: 1.7352x; 1.1221x over previous
import functools

import jax
import jax.numpy as jnp
from jax import lax
from jax.experimental import pallas as pl
from jax.experimental.pallas import tpu as pltpu
from jax.experimental.pallas import tpu_sc as plsc

N_FEATURES = 40960
D = 257
DH = 256
DP = DH // 2
BATCH = 16384
K = 32
NBAGS = 2 * BATCH
NC, NS = 2, 16
NW = NC * NS
BAGS_PER_W = NBAGS // NW
BCH = 128
NCH = BAGS_PER_W // BCH
GROWS = 128
GBAGS = GROWS // K
NG = BCH // GBAGS
L = 16


def _mesh():
    return plsc.VectorSubcoreMesh(
        core_axis_name="c", subcore_axis_name="s", num_cores=NC, num_subcores=NS
    )


def _pack_body(w_ref, out_ref, psqt_ref):
    x = w_ref[...]
    u = lax.bitcast_convert_type(x[:, :DH], jnp.uint32)
    r = (u + jnp.uint32(0x7FFF) + ((u >> 16) & jnp.uint32(1))) >> 16
    lo, hi = r[:, :DP], r[:, DP:]
    out_ref[...] = lax.bitcast_convert_type(lo | (hi << 16), jnp.int32)
    psqt_ref[...] = x[:, DH:]


@functools.cache
def _make_pack():
    blk = 4096
    return pl.pallas_call(
        _pack_body,
        grid=(N_FEATURES // blk,),
        in_specs=[pl.BlockSpec((blk, D), lambda i: (i, 0))],
        out_specs=[
            pl.BlockSpec((blk, DP), lambda i: (i, 0)),
            pl.BlockSpec((blk, 1), lambda i: (i, 0)),
        ],
        out_shape=[
            jax.ShapeDtypeStruct((N_FEATURES, DP), jnp.int32),
            jax.ShapeDtypeStruct((N_FEATURES, 1), jnp.float32),
        ],
    )


@functools.cache
def _make_embed_bag():
    return functools.partial(
        pl.kernel,
        out_type=(
            jax.ShapeDtypeStruct((NBAGS, DP), jnp.int32),
            jax.ShapeDtypeStruct((NBAGS,), jnp.float32),
        ),
        mesh=_mesh(),
        compiler_params=pltpu.CompilerParams(needs_layout_passes=False),
        scratch_types=[
            pltpu.VMEM((BCH * K,), jnp.int32),
            pltpu.VMEM((4, GROWS, DP), jnp.int32),
            pltpu.VMEM((BCH, DP), jnp.int32),
            pltpu.VMEM((N_FEATURES,), jnp.float32),
            pltpu.VMEM((BCH,), jnp.float32),
            pltpu.SemaphoreType.DMA,
            pltpu.SemaphoreType.DMA,
            pltpu.SemaphoreType.DMA,
            pltpu.SemaphoreType.DMA,
        ],
    )(_embed_bag_body)


def _embed_bag_body(wics_hbm, bics_hbm, table_hbm, psqt_hbm,
                    out_hbm, outp_hbm,
                    idxf, rows, acc, psqt_v, pacc, sem0, sem1, sem2, sem3):
    wid = lax.axis_index("s") * NC + lax.axis_index("c")
    pltpu.sync_copy(psqt_hbm, psqt_v)
    sems = (sem0, sem1, sem2, sem3)
    NB = 4

    def fire(g, buf):
        pltpu.async_copy(
            table_hbm.at[idxf.at[pl.ds(g * GROWS, GROWS)]],
            rows.at[buf], sems[buf])

    def wait(buf):
        pltpu.make_async_copy(
            table_hbm.at[idxf.at[pl.ds(0, GROWS)]],
            rows.at[buf], sems[buf]).wait()

    def reduce_buf(g, buf):
        rb = rows.at[buf]

        def ld(r, s):
            return plsc.bitcast(rb[r, s], jnp.bfloat16)

        def bag_body(bag, c1):
            r0 = bag * K
            for cb in range(DP // L):
                s = pl.ds(cb * L, L)
                a = [ld(r0 + i, s) + ld(r0 + 8 + i, s) for i in range(8)]
                for i in range(8):
                    a[i] = a[i] + ld(r0 + 16 + i, s)
                    a[i] = a[i] + ld(r0 + 24 + i, s)
                b0 = (a[0] + a[1]) + (a[2] + a[3])
                b1 = (a[4] + a[5]) + (a[6] + a[7])
                acc[g * GBAGS + bag, s] = plsc.bitcast(b0 + b1, jnp.int32)
            return c1

        lax.fori_loop(0, GBAGS, bag_body, 0)

    def chunk_body(ci, carry):
        base = wid * BAGS_PER_W + ci * BCH
        lbase = (wid % NS) * BAGS_PER_W * K + ci * BCH * K

        @pl.when(wid < NS)
        def _():
            pltpu.sync_copy(wics_hbm.at[pl.ds(lbase, BCH * K)], idxf)

        @pl.when(wid >= NS)
        def _():
            pltpu.sync_copy(bics_hbm.at[pl.ds(lbase, BCH * K)], idxf)

        for b in range(4):
            fire(b, b)

        lanes = lax.iota(jnp.int32, 16)

        def psum(blk, c1):
            vacc = jnp.zeros((L,), jnp.float32)
            for lane in range(L):
                p0 = (blk * L + lane) * K
                v = plsc.load_gather(psqt_v, [idxf[pl.ds(p0, L)]])
                v = v + plsc.load_gather(psqt_v, [idxf[pl.ds(p0 + L, L)]])
                s = lax.reduce_sum(v, axes=(0,))
                vacc = jnp.where(lanes == lane, s, vacc)
            pacc[pl.ds(blk * L, L)] = vacc
            return c1

        lax.fori_loop(0, BCH // L, psum, 0)

        def pipe_body(g4, c1):
            g = 4 * g4
            for b in range(4):
                wait(b)
                reduce_buf(g + b, b)

                @pl.when(g4 != NG // 4 - 1)
                def _():
                    fire(g + b + 4, b)

            return c1

        lax.fori_loop(0, NG // 4, pipe_body, 0)
        pltpu.sync_copy(acc, out_hbm.at[pl.ds(base, BCH)])
        pltpu.sync_copy(pacc, outp_hbm.at[pl.ds(base, BCH)])
        return carry

    lax.fori_loop(0, NCH, chunk_body, 0)


def _fc_body(w_ref, b_ref, wp_ref, bp_ref, stm_ref, bias_ref, fcw_ref,
             fcb_ref, out_ref):
    def unpack(v32):
        lo = lax.bitcast_convert_type(v32 << 16, jnp.float32)
        hi = lax.bitcast_convert_type(
            v32 & jnp.int32(-65536), jnp.float32)
        return jnp.concatenate([lo, hi], axis=1)

    bias = bias_ref[...]
    wfts = unpack(w_ref[...]) + bias
    bfts = unpack(b_ref[...]) + bias
    s = stm_ref[...]
    x1 = (1.0 - s) * wfts + s * bfts
    x2 = (1.0 - s) * bfts + s * wfts
    fcw = fcw_ref[...]
    fca, fcbb = fcw[:, :DH], fcw[:, DH:]
    acc = jnp.sum(jnp.clip(x1, 0.0, 1.0) * fca, axis=1, keepdims=True)
    acc = acc + jnp.sum(jnp.clip(x2, 0.0, 1.0) * fcbb, axis=1, keepdims=True)
    out_ref[...] = acc + fcb_ref[...] + (wp_ref[...] - bp_ref[...]) * (0.5 - s)


def kernel(wft_ics, bft_ics, stm, ft_weight, ft_bias, fc_w, fc_b):
    t32, psqt_col = _make_pack()(ft_weight)
    acc32, psqt = _make_embed_bag()(
        wft_ics.reshape(-1), bft_ics.reshape(-1), t32, psqt_col.reshape(-1))
    psqt2 = psqt.reshape(NBAGS, 1)

    bm = 2048
    nb = BATCH // bm
    out = pl.pallas_call(
        _fc_body,
        grid=(nb,),
        in_specs=[
            pl.BlockSpec((bm, DP), lambda i: (i, 0)),
            pl.BlockSpec((bm, DP), lambda i: (i + nb, 0)),
            pl.BlockSpec((bm, 1), lambda i: (i, 0)),
            pl.BlockSpec((bm, 1), lambda i: (i + nb, 0)),
            pl.BlockSpec((bm, 1), lambda i: (i, 0)),
            pl.BlockSpec((1, DH), lambda i: (0, 0)),
            pl.BlockSpec((1, 512), lambda i: (0, 0)),
            pl.BlockSpec((1, 1), lambda i: (0, 0)),
        ],
        out_specs=pl.BlockSpec((bm, 1), lambda i: (i, 0)),
        out_shape=jax.ShapeDtypeStruct((BATCH, 1), jnp.float32),
    )(acc32, acc32, psqt2, psqt2, stm, ft_bias[:DH].reshape(1, DH), fc_w,
      fc_b.reshape(1, 1))
    return out
